# baseline (speedup 1.0000x reference)
import functools

import jax
import jax.numpy as jnp
from jax import lax
from jax.experimental import pallas as pl
from jax.experimental.pallas import tpu as pltpu
from jax.experimental.pallas import tpu_sc as plsc

NUM_USERS = 1000000
NUM_MOVIES = 100000
EMB = 32
B = 16384

NC, NS = 2, 16
NW = NC * NS
BPW = B // NW
CHUNK = 128
NCHUNK = BPW // CHUNK

BB = 2048


def _sc_gather_body(uidx_hbm, midx_hbm, user_table, movie_table,
                    u_out, m_out, uidx_v, midx_v, urows_v, mrows_v, sem):
    wid = lax.axis_index("s") * NC + lax.axis_index("c")
    row0 = wid * NCHUNK
    pltpu.sync_copy(uidx_hbm.at[pl.ds(row0, NCHUNK)], uidx_v)
    pltpu.sync_copy(midx_hbm.at[pl.ds(row0, NCHUNK)], midx_v)
    copies = []
    for j in range(NCHUNK):
        copies.append(pltpu.async_copy(
            user_table.at[uidx_v.at[j]],
            urows_v.at[pl.ds(j * CHUNK, CHUNK)], sem))
        copies.append(pltpu.async_copy(
            movie_table.at[midx_v.at[j]],
            mrows_v.at[pl.ds(j * CHUNK, CHUNK)], sem))
    for c in copies:
        c.wait()
    base = wid * BPW
    pltpu.sync_copy(urows_v, u_out.at[pl.ds(base, BPW)])
    pltpu.sync_copy(mrows_v, m_out.at[pl.ds(base, BPW)])


@functools.cache
def _sc_gather():
    return pl.kernel(
        _sc_gather_body,
        out_type=(jax.ShapeDtypeStruct((B, EMB), jnp.float32),
                  jax.ShapeDtypeStruct((B, EMB), jnp.float32)),
        mesh=plsc.VectorSubcoreMesh(core_axis_name="c", subcore_axis_name="s"),
        scratch_types=(
            pltpu.VMEM((NCHUNK, CHUNK), jnp.int32),
            pltpu.VMEM((NCHUNK, CHUNK), jnp.int32),
            pltpu.VMEM((BPW, EMB), jnp.float32),
            pltpu.VMEM((BPW, EMB), jnp.float32),
            pltpu.SemaphoreType.DMA,
        ),
        compiler_params=pltpu.CompilerParams(use_tc_tiling_on_sc=False),
    )


def _mlp_body(u_ref, m_ref, g_ref, l_ref, vc_ref, va_ref,
              w1u_ref, w1m_ref, w1r_ref, b1_ref, w2_ref, b2_ref,
              w3_ref, b3_ref, out_ref):
    f32 = jnp.float32
    h1 = jnp.dot(u_ref[...], w1u_ref[...], preferred_element_type=f32)
    h1 += jnp.dot(m_ref[...], w1m_ref[...], preferred_element_type=f32)
    fs = jnp.concatenate(
        [g_ref[...], l_ref[...], vc_ref[...], va_ref[...]], axis=1)
    h1 += jnp.dot(fs, w1r_ref[...], preferred_element_type=f32)
    h1 = jnp.maximum(h1 + b1_ref[...], 0.0)
    h2 = jnp.maximum(
        jnp.dot(h1, w2_ref[...], preferred_element_type=f32) + b2_ref[...],
        0.0)
    out_ref[...] = (jnp.sum(h2 * w3_ref[...], axis=1, keepdims=True)
                    + b3_ref[...])


def _full(shape):
    return pl.BlockSpec(shape, lambda i: (0, 0))


_mlp = pl.pallas_call(
    _mlp_body,
    grid=(B // BB,),
    in_specs=[
        pl.BlockSpec((BB, EMB), lambda i: (i, 0)),
        pl.BlockSpec((BB, EMB), lambda i: (i, 0)),
        pl.BlockSpec((BB, 20), lambda i: (i, 0)),
        pl.BlockSpec((BB, 5), lambda i: (i, 0)),
        pl.BlockSpec((BB, 1), lambda i: (i, 0)),
        pl.BlockSpec((BB, 1), lambda i: (i, 0)),
        _full((EMB, 64)),
        _full((EMB, 64)),
        _full((27, 64)),
        _full((1, 64)),
        _full((64, 32)),
        _full((1, 32)),
        _full((1, 32)),
        _full((1, 1)),
    ],
    out_specs=pl.BlockSpec((BB, 1), lambda i: (i, 0)),
    out_shape=jax.ShapeDtypeStruct((B, 1), jnp.float32),
)


def kernel(user, movie, genres, lang, vote_count, vote_avg,
           user_table, movie_table, W1, b1, W2, b2, W3, b3):
    uidx = user.astype(jnp.int32).reshape(B // CHUNK, CHUNK)
    midx = movie.astype(jnp.int32).reshape(B // CHUNK, CHUNK)
    u, m = _sc_gather()(uidx, midx, user_table, movie_table)
    return _mlp(u, m, genres, lang, vote_count, vote_avg,
                W1[:EMB], W1[EMB:2 * EMB], W1[2 * EMB:],
                b1.reshape(1, 64), W2, b2.reshape(1, 32),
                W3.reshape(1, 32), b3.reshape(1, 1))
